# baseline (speedup 1.0000x reference)
import functools

import jax
import jax.numpy as jnp
from jax.experimental import pallas as pl
from jax.experimental.pallas import tpu as pltpu

_B = 4
_E = 8
_K = 2


def _router_body(x_ref, w_ref, logits_ref, weights_ref, experts_ref, acc_ref,
                 *, nsteps):
    i = pl.program_id(0)

    @pl.when(i == 0)
    def _init():
        acc_ref[...] = jnp.zeros_like(acc_ref)

    acc_ref[...] += jax.lax.dot_general(
        x_ref[...], w_ref[...],
        dimension_numbers=(((1,), (1,)), ((), ())),
        preferred_element_type=jnp.float32,
    )

    @pl.when(i == nsteps - 1)
    def _epilogue():
        logits = acc_ref[...]
        logits_ref[...] = logits
        m = jnp.max(logits, axis=1, keepdims=True)
        p = jnp.exp(logits - m)
        w = p / jnp.sum(p, axis=1, keepdims=True)
        idx = jax.lax.broadcasted_iota(jnp.int32, (_B, _E), 1)
        neg = jnp.float32(-jnp.inf)
        m1 = jnp.max(w, axis=1, keepdims=True)
        i1 = jnp.min(jnp.where(w == m1, idx, _E), axis=1, keepdims=True)
        w2 = jnp.where(idx == i1, neg, w)
        m2 = jnp.max(w2, axis=1, keepdims=True)
        i2 = jnp.min(jnp.where(w2 == m2, idx, _E), axis=1, keepdims=True)
        s = m1 + m2
        weights_ref[...] = jnp.concatenate([m1 / s, m2 / s], axis=1)
        experts_ref[...] = jnp.concatenate([i1, i2], axis=1)


@jax.jit
def kernel(inputs, W):
    b = inputs.shape[0]
    x = inputs.reshape(b, -1)
    n = x.shape[1]
    e = W.shape[0]
    chunk = 256 * 1024
    nsteps = n // chunk

    body = functools.partial(_router_body, nsteps=nsteps)
    logits, weights, experts = pl.pallas_call(
        body,
        grid=(nsteps,),
        in_specs=[
            pl.BlockSpec((b, chunk), lambda i: (0, i)),
            pl.BlockSpec((e, chunk), lambda i: (0, i)),
        ],
        out_specs=[
            pl.BlockSpec((b, e), lambda i: (0, 0)),
            pl.BlockSpec((b, _K), lambda i: (0, 0)),
            pl.BlockSpec((b, _K), lambda i: (0, 0)),
        ],
        out_shape=[
            jax.ShapeDtypeStruct((b, e), jnp.float32),
            jax.ShapeDtypeStruct((b, _K), jnp.float32),
            jax.ShapeDtypeStruct((b, _K), jnp.int32),
        ],
        scratch_shapes=[pltpu.VMEM((b, e), jnp.float32)],
    )(x, W)
    return (weights, experts, logits)
